# baseline (speedup 1.0000x reference)
import jax
import jax.numpy as jnp
from jax.experimental import pallas as pl
from jax.experimental.pallas import tpu as pltpu

B, N, C, H, W = 4, 8, 384, 24, 24
BANK = 4
HW = H * W
BN = B * N
BLK = 4


def _pool_kernel(value_ref, mask_ref, z_ref):
    v = value_ref[0, 0]
    m = mask_ref[0, 0]
    denom = jnp.sum(m)
    coef = jnp.where(denom > 1e-5, m / jnp.maximum(denom, 1e-6), 1.0 / HW)
    z_ref[0, 0] = jnp.sum(v * coef[None], axis=(1, 2), keepdims=True)


def _retrieve_kernel(z_ref, keys_ref, vals_ref, gate_ref, out_ref):
    z = z_ref[...]
    keys = keys_ref[...]
    vals = vals_ref[...]
    diff = z - keys
    dist = jnp.sum(diff * diff, axis=2, keepdims=True)
    logits = -dist
    mx = jnp.max(logits, axis=1, keepdims=True)
    e = jnp.exp(logits - mx)
    wts = e / jnp.sum(e, axis=1, keepdims=True)
    readout = jnp.sum(wts * vals, axis=1, keepdims=True)
    out_ref[...] = z + gate_ref[0, 0] * readout


def kernel(value_BNCHW, key_BCHW, pixfeat_BCHW, mask_BNHW, bank_keys, bank_vals, gate):
    del key_BCHW, pixfeat_BCHW
    gate2 = jnp.asarray(gate, jnp.float32).reshape(1, 1)

    z_col = pl.pallas_call(
        _pool_kernel,
        grid=(B, N),
        in_specs=[
            pl.BlockSpec((1, 1, C, H, W), lambda b, n: (b, n, 0, 0, 0)),
            pl.BlockSpec((1, 1, H, W), lambda b, n: (b, n, 0, 0)),
        ],
        out_specs=pl.BlockSpec((1, 1, C, 1, 1), lambda b, n: (b, n, 0, 0, 0)),
        out_shape=jax.ShapeDtypeStruct((B, N, C, 1, 1), jnp.float32),
        compiler_params=pltpu.CompilerParams(
            dimension_semantics=("parallel", "parallel"),
        ),
    )(value_BNCHW, mask_BNHW)

    z3 = z_col.reshape(BN, 1, C)
    out = pl.pallas_call(
        _retrieve_kernel,
        in_specs=[
            pl.BlockSpec((BN, 1, C), lambda: (0, 0, 0)),
            pl.BlockSpec((BN, BANK, C), lambda: (0, 0, 0)),
            pl.BlockSpec((BN, BANK, C), lambda: (0, 0, 0)),
            pl.BlockSpec((1, 1), lambda: (0, 0)),
        ],
        out_specs=pl.BlockSpec((BN, 1, C), lambda: (0, 0, 0)),
        out_shape=jax.ShapeDtypeStruct((BN, 1, C), jnp.float32),
    )(z3, bank_keys.reshape(BN, BANK, C), bank_vals.reshape(BN, BANK, C), gate2)
    return out.reshape(B, N, C)


# speedup vs baseline: 1.1424x; 1.1424x over previous
import jax
import jax.numpy as jnp
from jax.experimental import pallas as pl
from jax.experimental.pallas import tpu as pltpu

B, N, C, H, W = 4, 8, 384, 24, 24
BANK = 4
HW = H * W
BN = B * N
ROWS = C * HW // 128
CHUNK = 4
CROWS = CHUNK * ROWS
PLANES = CROWS // 72
CPP = 16


def _pool_kernel(value_any, pat_ref, e_ref, al_ref, ar_ref, z_ref, vbuf, sem):
    i = pl.program_id(0)
    cp = pltpu.make_async_copy(value_any.at[i], vbuf, sem)
    cp.start()
    cp.wait()
    v = vbuf[...].reshape(PLANES, 72, 128)
    pat = pat_ref[...]
    coef = jnp.broadcast_to(
        pat[:, None], (CHUNK, PLANES // CHUNK, 72, 128)
    ).reshape(PLANES, 72, 128)
    g = v * coef
    hp = jax.lax.dot_general(
        e_ref[...], g, (((1,), (2,)), ((), ())),
        preferred_element_type=jnp.float32,
    )
    zl = jax.lax.dot_general(
        hp[0], al_ref[...], (((1,), (0,)), ((), ())),
        preferred_element_type=jnp.float32,
    )
    zr = jax.lax.dot_general(
        hp[1], ar_ref[...], (((1,), (0,)), ((), ())),
        preferred_element_type=jnp.float32,
    )
    z_ref[...] = zl + zr


def _retrieve_kernel(z_ref, keys_ref, vals_ref, gate_ref, out_ref):
    z = z_ref[...]
    keys = keys_ref[...]
    vals = vals_ref[...]
    diff = z - keys
    dist = jnp.sum(diff * diff, axis=2, keepdims=True)
    logits = -dist
    mx = jnp.max(logits, axis=1, keepdims=True)
    e = jnp.exp(logits - mx)
    wts = e / jnp.sum(e, axis=1, keepdims=True)
    readout = jnp.sum(wts * vals, axis=1, keepdims=True)
    out_ref[...] = z + gate_ref[0, 0] * readout


def kernel(value_BNCHW, key_BCHW, pixfeat_BCHW, mask_BNHW, bank_keys, bank_vals, gate):
    del key_BCHW, pixfeat_BCHW
    mask = mask_BNHW.reshape(BN, HW)
    denom = jnp.sum(mask, axis=1, keepdims=True)
    coef = jnp.where(denom > 1e-5, mask / jnp.maximum(denom, 1e-6), 1.0 / HW)
    pat = jnp.tile(coef, (1, CPP)).reshape(BN, 72, 128)
    lanes = jnp.arange(128)
    e_half = jnp.stack([(lanes < 64), (lanes >= 64)]).astype(jnp.float32)
    rows = jnp.arange(72)
    al_sel = ((rows * 128) // HW)[:, None] == jnp.arange(CPP)[None, :]
    ar_sel = ((rows * 128 + 64) // HW)[:, None] == jnp.arange(CPP)[None, :]
    al_sel = al_sel.astype(jnp.float32)
    ar_sel = ar_sel.astype(jnp.float32)
    gate2 = jnp.asarray(gate, jnp.float32).reshape(1, 1)

    z_flat = pl.pallas_call(
        _pool_kernel,
        grid=(BN // CHUNK,),
        in_specs=[
            pl.BlockSpec(memory_space=pl.ANY),
            pl.BlockSpec((CHUNK, 72, 128), lambda i: (i, 0, 0)),
            pl.BlockSpec((2, 128), lambda i: (0, 0)),
            pl.BlockSpec((72, CPP), lambda i: (0, 0)),
            pl.BlockSpec((72, CPP), lambda i: (0, 0)),
        ],
        out_specs=pl.BlockSpec((PLANES, CPP), lambda i: (i, 0)),
        out_shape=jax.ShapeDtypeStruct((BN * ROWS // 72, CPP), jnp.float32),
        scratch_shapes=[
            pltpu.VMEM((CROWS, 128), jnp.float32),
            pltpu.SemaphoreType.DMA,
        ],
        compiler_params=pltpu.CompilerParams(
            dimension_semantics=("arbitrary",),
        ),
    )(value_BNCHW.reshape(BN // CHUNK, CROWS, 128), pat, e_half, al_sel, ar_sel)

    z3 = z_flat.reshape(BN, 1, C)
    out = pl.pallas_call(
        _retrieve_kernel,
        in_specs=[
            pl.BlockSpec((BN, 1, C), lambda: (0, 0, 0)),
            pl.BlockSpec((BN, BANK, C), lambda: (0, 0, 0)),
            pl.BlockSpec((BN, BANK, C), lambda: (0, 0, 0)),
            pl.BlockSpec((1, 1), lambda: (0, 0)),
        ],
        out_specs=pl.BlockSpec((BN, 1, C), lambda: (0, 0, 0)),
        out_shape=jax.ShapeDtypeStruct((BN, 1, C), jnp.float32),
    )(z3, bank_keys.reshape(BN, BANK, C), bank_vals.reshape(BN, BANK, C), gate2)
    return out.reshape(B, N, C)


# speedup vs baseline: 1.2045x; 1.0544x over previous
import jax
import jax.numpy as jnp
from jax.experimental import pallas as pl
from jax.experimental.pallas import tpu as pltpu

B, N, C, H, W = 4, 8, 384, 24, 24
BANK = 4
HW = H * W
BN = B * N
ROWS = C * HW // 128
CHUNK = 4
CROWS = CHUNK * ROWS
PLANES = CROWS // 72
CPP = 16


def _pool_kernel(v_ref, pat_ref, e_ref, al_ref, ar_ref, z_ref):
    v = v_ref[...].reshape(PLANES, 72, 128)
    pat = pat_ref[...]
    coef = jnp.broadcast_to(
        pat[:, None], (CHUNK, PLANES // CHUNK, 72, 128)
    ).reshape(PLANES, 72, 128)
    g = v * coef
    hp = jax.lax.dot_general(
        e_ref[...], g, (((1,), (2,)), ((), ())),
        preferred_element_type=jnp.float32,
    )
    zl = jax.lax.dot_general(
        hp[0], al_ref[...], (((1,), (0,)), ((), ())),
        preferred_element_type=jnp.float32,
    )
    zr = jax.lax.dot_general(
        hp[1], ar_ref[...], (((1,), (0,)), ((), ())),
        preferred_element_type=jnp.float32,
    )
    z_ref[...] = zl + zr


def _retrieve_kernel(z_ref, keys_ref, vals_ref, gate_ref, out_ref):
    z = z_ref[...]
    keys = keys_ref[...]
    vals = vals_ref[...]
    diff = z - keys
    dist = jnp.sum(diff * diff, axis=2, keepdims=True)
    logits = -dist
    mx = jnp.max(logits, axis=1, keepdims=True)
    e = jnp.exp(logits - mx)
    wts = e / jnp.sum(e, axis=1, keepdims=True)
    readout = jnp.sum(wts * vals, axis=1, keepdims=True)
    out_ref[...] = z + gate_ref[0, 0] * readout


def kernel(value_BNCHW, key_BCHW, pixfeat_BCHW, mask_BNHW, bank_keys, bank_vals, gate):
    del key_BCHW, pixfeat_BCHW
    mask = mask_BNHW.reshape(BN, HW)
    denom = jnp.sum(mask, axis=1, keepdims=True)
    coef = jnp.where(denom > 1e-5, mask / jnp.maximum(denom, 1e-6), 1.0 / HW)
    pat = jnp.tile(coef, (1, CPP)).reshape(BN, 72, 128)
    lanes = jnp.arange(128)
    e_half = jnp.stack([(lanes < 64), (lanes >= 64)]).astype(jnp.float32)
    rows = jnp.arange(72)
    al_sel = ((rows * 128) // HW)[:, None] == jnp.arange(CPP)[None, :]
    ar_sel = ((rows * 128 + 64) // HW)[:, None] == jnp.arange(CPP)[None, :]
    al_sel = al_sel.astype(jnp.float32)
    ar_sel = ar_sel.astype(jnp.float32)
    gate2 = jnp.asarray(gate, jnp.float32).reshape(1, 1)

    z_flat = pl.pallas_call(
        _pool_kernel,
        grid=(BN // CHUNK,),
        in_specs=[
            pl.BlockSpec((1, CROWS, 128), lambda i: (i, 0, 0)),
            pl.BlockSpec((CHUNK, 72, 128), lambda i: (i, 0, 0)),
            pl.BlockSpec((2, 128), lambda i: (0, 0)),
            pl.BlockSpec((72, CPP), lambda i: (0, 0)),
            pl.BlockSpec((72, CPP), lambda i: (0, 0)),
        ],
        out_specs=pl.BlockSpec((PLANES, CPP), lambda i: (i, 0)),
        out_shape=jax.ShapeDtypeStruct((BN * ROWS // 72, CPP), jnp.float32),
        compiler_params=pltpu.CompilerParams(
            dimension_semantics=("arbitrary",),
        ),
    )(value_BNCHW.reshape(BN // CHUNK, CROWS, 128), pat, e_half, al_sel, ar_sel)

    z3 = z_flat.reshape(BN, 1, C)
    out = pl.pallas_call(
        _retrieve_kernel,
        in_specs=[
            pl.BlockSpec((BN, 1, C), lambda: (0, 0, 0)),
            pl.BlockSpec((BN, BANK, C), lambda: (0, 0, 0)),
            pl.BlockSpec((BN, BANK, C), lambda: (0, 0, 0)),
            pl.BlockSpec((1, 1), lambda: (0, 0)),
        ],
        out_specs=pl.BlockSpec((BN, 1, C), lambda: (0, 0, 0)),
        out_shape=jax.ShapeDtypeStruct((BN, 1, C), jnp.float32),
    )(z3, bank_keys.reshape(BN, BANK, C), bank_vals.reshape(BN, BANK, C), gate2)
    return out.reshape(B, N, C)


# speedup vs baseline: 7.8270x; 6.4980x over previous
import jax
import jax.numpy as jnp
from jax.experimental import pallas as pl
from jax.experimental.pallas import tpu as pltpu

B, N, C, H, W = 4, 8, 384, 24, 24
BANK = 4
HW = H * W
BN = B * N


def _fused_kernel(v_ref, m_ref, keys_ref, vals_ref, gate_ref, out_ref):
    m = m_ref[0]
    s = jnp.sum(m)
    coef = jnp.where(s > 1e-5, m / jnp.maximum(s, 1e-6),
                     jnp.full_like(m, 1.0 / HW))
    z = jax.lax.dot_general(
        coef, v_ref[0], (((1,), (0,)), ((), ())),
        preferred_element_type=jnp.float32,
    )
    keys = keys_ref[0]
    vals = vals_ref[0]
    diff = z - keys
    dist = jnp.sum(diff * diff, axis=1, keepdims=True)
    logits = -dist
    mx = jnp.max(logits, axis=0, keepdims=True)
    e = jnp.exp(logits - mx)
    wts = e / jnp.sum(e, axis=0, keepdims=True)
    readout = jnp.sum(wts * vals, axis=0, keepdims=True)
    out_ref[0] = z + gate_ref[0, 0] * readout


def kernel(value_BNCHW, key_BCHW, pixfeat_BCHW, mask_BNHW, bank_keys, bank_vals, gate):
    del key_BCHW, pixfeat_BCHW
    v = jnp.transpose(value_BNCHW, (0, 1, 3, 4, 2)).reshape(BN, HW, C)
    mask = mask_BNHW.reshape(BN, 1, HW)
    gate2 = jnp.asarray(gate, jnp.float32).reshape(1, 1)
    out = pl.pallas_call(
        _fused_kernel,
        grid=(BN,),
        in_specs=[
            pl.BlockSpec((1, HW, C), lambda i: (i, 0, 0)),
            pl.BlockSpec((1, 1, HW), lambda i: (i, 0, 0)),
            pl.BlockSpec((1, BANK, C), lambda i: (i, 0, 0)),
            pl.BlockSpec((1, BANK, C), lambda i: (i, 0, 0)),
            pl.BlockSpec((1, 1), lambda i: (0, 0)),
        ],
        out_specs=pl.BlockSpec((1, 1, C), lambda i: (i, 0, 0)),
        out_shape=jax.ShapeDtypeStruct((BN, 1, C), jnp.float32),
        compiler_params=pltpu.CompilerParams(
            dimension_semantics=("arbitrary",),
        ),
    )(v, mask, bank_keys.reshape(BN, BANK, C), bank_vals.reshape(BN, BANK, C), gate2)
    return out.reshape(B, N, C)


# speedup vs baseline: 9.4861x; 1.2120x over previous
import jax
import jax.numpy as jnp
from jax.experimental import pallas as pl
from jax.experimental.pallas import tpu as pltpu

B, N, C, H, W = 4, 8, 384, 24, 24
BANK = 4
HW = H * W
BN = B * N
CHUNK = 2
STEPS = BN // CHUNK


def _fused_kernel(v_ref, coef_ref, keys_ref, vals_ref, gate_ref, out_ref):
    for k in range(CHUNK):
        coef = coef_ref[0, pl.ds(k, 1)]
        z = jax.lax.dot_general(
            coef, v_ref[k], (((1,), (0,)), ((), ())),
            preferred_element_type=jnp.float32,
        )
        keys = keys_ref[0, pl.ds(k * BANK, BANK)]
        vals = vals_ref[0, pl.ds(k * BANK, BANK)]
        diff = z - keys
        dist = jnp.sum(diff * diff, axis=1, keepdims=True)
        logits = -dist
        mx = jnp.max(logits, axis=0, keepdims=True)
        e = jnp.exp(logits - mx)
        wts = e / jnp.sum(e, axis=0, keepdims=True)
        readout = jnp.sum(wts * vals, axis=0, keepdims=True)
        out_ref[0, pl.ds(k, 1)] = z + gate_ref[0, 0] * readout


def kernel(value_BNCHW, key_BCHW, pixfeat_BCHW, mask_BNHW, bank_keys, bank_vals, gate):
    del key_BCHW, pixfeat_BCHW
    v = jnp.transpose(value_BNCHW, (0, 1, 3, 4, 2)).reshape(BN, HW, C)
    mask = mask_BNHW.reshape(BN, HW)
    denom = jnp.sum(mask, axis=1, keepdims=True)
    coef = jnp.where(denom > 1e-5, mask / jnp.maximum(denom, 1e-6), 1.0 / HW)
    gate2 = jnp.asarray(gate, jnp.float32).reshape(1, 1)
    out = pl.pallas_call(
        _fused_kernel,
        grid=(STEPS,),
        in_specs=[
            pl.BlockSpec((CHUNK, HW, C), lambda i: (i, 0, 0)),
            pl.BlockSpec((1, CHUNK, HW), lambda i: (i, 0, 0)),
            pl.BlockSpec((1, CHUNK * BANK, C), lambda i: (i, 0, 0)),
            pl.BlockSpec((1, CHUNK * BANK, C), lambda i: (i, 0, 0)),
            pl.BlockSpec((1, 1), lambda i: (0, 0)),
        ],
        out_specs=pl.BlockSpec((1, CHUNK, C), lambda i: (i, 0, 0)),
        out_shape=jax.ShapeDtypeStruct((STEPS, CHUNK, C), jnp.float32),
        compiler_params=pltpu.CompilerParams(
            dimension_semantics=("arbitrary",),
        ),
    )(v, coef.reshape(STEPS, CHUNK, HW),
      bank_keys.reshape(STEPS, CHUNK * BANK, C),
      bank_vals.reshape(STEPS, CHUNK * BANK, C), gate2)
    return out.reshape(B, N, C)


# speedup vs baseline: 9.9489x; 1.0488x over previous
import jax
import jax.numpy as jnp
from jax.experimental import pallas as pl
from jax.experimental.pallas import tpu as pltpu

B, N, C, H, W = 4, 8, 384, 24, 24
BANK = 4
HW = H * W
BN = B * N
CHUNK = 2
STEPS = BN // CHUNK


def _fused_kernel(v_ref, mask_ref, kv_ref, gate_ref, out_ref):
    i = pl.program_id(0)
    kv = kv_ref[0, 0, pl.ds(i, 1)]
    vv = kv_ref[0, 1, pl.ds(i, 1)]
    for k in range(CHUNK):
        m = mask_ref[pl.ds(i, 1), k]
        s = jnp.sum(m)
        coef = jnp.where(s > 1e-5, m / jnp.maximum(s, 1e-6),
                         jnp.full_like(m, 1.0 / HW))
        z = jax.lax.dot_general(
            coef, v_ref[k], (((1,), (0,)), ((), ())),
            preferred_element_type=jnp.float32,
        )
        keys = kv[0, k * BANK:(k + 1) * BANK]
        vals = vv[0, k * BANK:(k + 1) * BANK]
        diff = z - keys
        dist = jnp.sum(diff * diff, axis=1, keepdims=True)
        logits = -dist
        mx = jnp.max(logits, axis=0, keepdims=True)
        e = jnp.exp(logits - mx)
        wts = e / jnp.sum(e, axis=0, keepdims=True)
        readout = jnp.sum(wts * vals, axis=0, keepdims=True)
        out_ref[0, pl.ds(k, 1)] = z + gate_ref[0, 0] * readout


def kernel(value_BNCHW, key_BCHW, pixfeat_BCHW, mask_BNHW, bank_keys, bank_vals, gate):
    del key_BCHW, pixfeat_BCHW
    v = jnp.transpose(value_BNCHW, (0, 1, 3, 4, 2)).reshape(BN, HW, C)
    mask = mask_BNHW.reshape(STEPS, CHUNK, HW)
    kv = jnp.stack([bank_keys, bank_vals]).reshape(1, 2, STEPS, CHUNK * BANK, C)
    gate2 = jnp.asarray(gate, jnp.float32).reshape(1, 1)
    out = pl.pallas_call(
        _fused_kernel,
        grid=(STEPS,),
        in_specs=[
            pl.BlockSpec((CHUNK, HW, C), lambda i: (i, 0, 0)),
            pl.BlockSpec((STEPS, CHUNK, HW), lambda i: (0, 0, 0)),
            pl.BlockSpec((1, 2, STEPS, CHUNK * BANK, C), lambda i: (0, 0, 0, 0, 0)),
            pl.BlockSpec((1, 1), lambda i: (0, 0)),
        ],
        out_specs=pl.BlockSpec((1, CHUNK, C), lambda i: (i, 0, 0)),
        out_shape=jax.ShapeDtypeStruct((STEPS, CHUNK, C), jnp.float32),
        compiler_params=pltpu.CompilerParams(
            dimension_semantics=("arbitrary",),
        ),
    )(v, mask, kv, gate2)
    return out.reshape(B, N, C)
